# Initial kernel scaffold; baseline (speedup 1.0000x reference)
import functools
import jax
import jax.numpy as jnp
from jax.experimental import pallas as pl
from jax.experimental.pallas import tpu as pltpu
from jax.experimental.pallas import tpu_sc as plsc

N_NODES = 10000
N_EDGES = 160000
NR = 1024
R = 10
N_PAD = NR * R
G = 32
E_PAD = N_EDGES + G + 8
BR = 400


def _sc_segsum(values, src_s, dst_s, roff, zeros_blk, feat):
    drain_rows = NR // 16

    mesh = plsc.VectorSubcoreMesh(core_axis_name="c", subcore_axis_name="s")

    @functools.partial(
        pl.kernel,
        out_type=jax.ShapeDtypeStruct((N_PAD, feat), jnp.float32),
        mesh=mesh,
        scratch_types=[
            pltpu.VMEM_SHARED((NR + 8, feat), jnp.float32),
            pltpu.VMEM((G, feat), jnp.float32),
            pltpu.VMEM((G,), jnp.int32),
            pltpu.VMEM((G,), jnp.int32),
            pltpu.SMEM((R + 1,), jnp.int32),
        ],
    )
    def k(vals_hbm, src_hbm, dst_hbm, roff_hbm, zero_hbm, out_hbm,
          acc, gbuf, sidx, didx, roffs):
        core = jax.lax.axis_index("c")
        sid = jax.lax.axis_index("s")
        pltpu.sync_copy(roff_hbm.at[:, 0], roffs)

        @pl.loop(0, R // 2)
        def _range(i):
            r = 2 * i + core
            nbase = r * NR
            t0all = roffs[r]
            t1all = roffs[r + 1]
            ln = t1all - t0all
            t0 = t0all + (ln * sid) // 16
            t1 = t0all + (ln * (sid + 1)) // 16

            pltpu.sync_copy(zero_hbm, acc.at[pl.ds(sid * drain_rows, drain_rows)])
            plsc.subcore_barrier()

            e0 = (t0 // 8) * 8
            nb = jnp.maximum((t1 - e0 + G - 1) // G, 0)

            @pl.loop(0, nb)
            def _blk(b):
                e = e0 + b * G
                pltpu.sync_copy(src_hbm.at[pl.ds(e, G)], sidx)
                pltpu.sync_copy(dst_hbm.at[pl.ds(e, G)], didx)
                for kk in range(0, G, 16):
                    pos = jax.lax.iota(jnp.int32, 16) + (e + kk)
                    m = (pos >= t0) & (pos < t1)
                    sv = sidx[pl.ds(kk, 16)]
                    dv = didx[pl.ds(kk, 16)]
                    sidx[pl.ds(kk, 16)] = jnp.where(m, sv, 0)
                    didx[pl.ds(kk, 16)] = jnp.where(m, dv - nbase, NR)
                pltpu.sync_copy(vals_hbm.at[sidx], gbuf)
                pltpu.sync_copy(gbuf, acc.at[didx], add=True)

            plsc.subcore_barrier()
            pltpu.sync_copy(
                acc.at[pl.ds(sid * drain_rows, drain_rows)],
                out_hbm.at[pl.ds(nbase + sid * drain_rows, drain_rows)],
            )
            plsc.subcore_barrier()

    return k(values, src_s, dst_s, roff, zeros_blk)


def _layer_tc(aggsum, xin, Wl, bl, Wr, off_lo, off_hi, want_relu, two_out):
    n, cin = xin.shape
    cout = Wl.shape[1]
    grid = n // BR

    def body(a_ref, x_ref, wl_ref, bl_ref, wr_ref, lo_ref, hi_ref, *outs):
        cnt = (hi_ref[...] - lo_ref[...]).astype(jnp.float32)
        inv = 1.0 / jnp.maximum(cnt, 1.0)
        a = a_ref[...] * inv
        out = (jnp.dot(a, wl_ref[...], preferred_element_type=jnp.float32)
               + bl_ref[...]
               + jnp.dot(x_ref[...], wr_ref[...],
                         preferred_element_type=jnp.float32))
        if two_out:
            outs[0][...] = out
            outs[1][...] = jnp.maximum(out, 0.0)
        elif want_relu:
            outs[0][...] = jnp.maximum(out, 0.0)
        else:
            outs[0][...] = out

    out_shape = jax.ShapeDtypeStruct((n, cout), jnp.float32)
    out_shapes = (out_shape, out_shape) if two_out else out_shape
    out_spec = pl.BlockSpec((BR, cout), lambda i: (i, 0))
    out_specs = (out_spec, out_spec) if two_out else out_spec

    return pl.pallas_call(
        body,
        grid=(grid,),
        in_specs=[
            pl.BlockSpec((BR, cin), lambda i: (i, 0)),
            pl.BlockSpec((BR, cin), lambda i: (i, 0)),
            pl.BlockSpec((cin, cout), lambda i: (0, 0)),
            pl.BlockSpec((1, cout), lambda i: (0, 0)),
            pl.BlockSpec((cin, cout), lambda i: (0, 0)),
            pl.BlockSpec((BR, 1), lambda i: (i, 0)),
            pl.BlockSpec((BR, 1), lambda i: (i, 0)),
        ],
        out_specs=out_specs,
        out_shape=out_shapes,
    )(aggsum, xin, Wl, bl, Wr, off_lo, off_hi)


def _mm2_tc(h2, Wl2p, Wr2p):
    n, cin = h2.shape
    cout = Wl2p.shape[1]
    grid = n // BR

    def body(h_ref, wl_ref, wr_ref, y_ref, z_ref):
        h = h_ref[...]
        y_ref[...] = jnp.dot(h, wl_ref[...], preferred_element_type=jnp.float32)
        z_ref[...] = jnp.dot(h, wr_ref[...], preferred_element_type=jnp.float32)

    shp = jax.ShapeDtypeStruct((n, cout), jnp.float32)
    spec = pl.BlockSpec((BR, cout), lambda i: (i, 0))
    return pl.pallas_call(
        body,
        grid=(grid,),
        in_specs=[
            pl.BlockSpec((BR, cin), lambda i: (i, 0)),
            pl.BlockSpec((cin, cout), lambda i: (0, 0)),
            pl.BlockSpec((cin, cout), lambda i: (0, 0)),
        ],
        out_specs=(spec, spec),
        out_shape=(shp, shp),
    )(h2, Wl2p, Wr2p)


def _final_tc(agg2sum, z2, bl2p, off_lo, off_hi, valid):
    n, cout = z2.shape
    grid = n // BR

    def body(a_ref, z_ref, bl_ref, lo_ref, hi_ref, logits_ref, logp_ref):
        cnt = (hi_ref[...] - lo_ref[...]).astype(jnp.float32)
        inv = 1.0 / jnp.maximum(cnt, 1.0)
        logits = a_ref[...] * inv + bl_ref[...] + z_ref[...]
        logits_ref[...] = logits
        col = jax.lax.broadcasted_iota(jnp.int32, (BR, cout), 1)
        mask = col < valid
        neg = jnp.float32(-1e30)
        lm = jnp.where(mask, logits, neg)
        mx = jnp.max(lm, axis=1, keepdims=True)
        ex = jnp.where(mask, jnp.exp(logits - mx), 0.0)
        lse = jnp.log(jnp.sum(ex, axis=1, keepdims=True))
        logp_ref[...] = logits - mx - lse

    shp = jax.ShapeDtypeStruct((n, cout), jnp.float32)
    spec = pl.BlockSpec((BR, cout), lambda i: (i, 0))
    return pl.pallas_call(
        body,
        grid=(grid,),
        in_specs=[
            pl.BlockSpec((BR, cout), lambda i: (i, 0)),
            pl.BlockSpec((BR, cout), lambda i: (i, 0)),
            pl.BlockSpec((1, cout), lambda i: (0, 0)),
            pl.BlockSpec((BR, 1), lambda i: (i, 0)),
            pl.BlockSpec((BR, 1), lambda i: (i, 0)),
        ],
        out_specs=(spec, spec),
        out_shape=(shp, shp),
    )(agg2sum, z2, bl2p, off_lo, off_hi)


def kernel(x, edge_index, Wl0, bl0, Wr0, Wl1, bl1, Wr1, Wl2, bl2, Wr2):
    src = edge_index[0]
    dst = edge_index[1]

    order = jnp.argsort(dst)
    dst_s = dst[order]
    src_s = src[order]
    pad = E_PAD - N_EDGES
    dst_sp = jnp.concatenate([dst_s, jnp.zeros((pad,), jnp.int32)])
    src_sp = jnp.concatenate([src_s, jnp.zeros((pad,), jnp.int32)])
    offsets = jnp.searchsorted(dst_s, jnp.arange(N_NODES + 1, dtype=jnp.int32))
    offsets = offsets.astype(jnp.int32)
    roff = jnp.searchsorted(
        dst_s, jnp.arange(R + 1, dtype=jnp.int32) * NR).astype(jnp.int32)
    roff = roff.reshape(R + 1, 1)
    off_lo = offsets[:N_NODES].reshape(N_NODES, 1)
    off_hi = offsets[1:].reshape(N_NODES, 1)

    zeros256 = jnp.zeros((NR // 16, 256), jnp.float32)
    zeros512 = jnp.zeros((NR // 16, 512), jnp.float32)
    zeros64 = jnp.zeros((NR // 16, 64), jnp.float32)

    bl0r = bl0.reshape(1, -1)
    bl1r = bl1.reshape(1, -1)
    Wl2p = jnp.pad(Wl2, ((0, 0), (0, 64 - 47)))
    Wr2p = jnp.pad(Wr2, ((0, 0), (0, 64 - 47)))
    bl2p = jnp.pad(bl2, (0, 64 - 47)).reshape(1, -1)

    agg0 = _sc_segsum(x, src_sp, dst_sp, roff, zeros256, 256)
    h = _layer_tc(agg0[:N_NODES], x, Wl0, bl0r, Wr0, off_lo, off_hi,
                  want_relu=True, two_out=False)

    agg1 = _sc_segsum(h, src_sp, dst_sp, roff, zeros512, 512)
    out2, h2 = _layer_tc(agg1[:N_NODES], h, Wl1, bl1r, Wr1, off_lo, off_hi,
                         want_relu=True, two_out=True)

    y2, z2 = _mm2_tc(h2, Wl2p, Wr2p)
    agg2 = _sc_segsum(y2, src_sp, dst_sp, roff, zeros64, 64)
    logits_p, logp_p = _final_tc(agg2[:N_NODES], z2, bl2p, off_lo, off_hi, 47)

    logits = logits_p[:, :47]
    logp = logp_p[:, :47]
    return (logp, out2, h2, logits)


# speedup vs baseline: 1.3800x; 1.3800x over previous
import dataclasses
import functools
import jax
import jax.numpy as jnp
from jax.experimental import pallas as pl
from jax.experimental.pallas import tpu as pltpu
from jax.experimental.pallas import tpu_sc as plsc

N_NODES = 10000
N_EDGES = 160000
NR = 1024
R = 10
N_PAD = NR * R
GE = 128
G = 128
E_NEW = N_EDGES + R * GE
BR = 400


def _sc_segsum(values2d, src_x, scat_x, broff, zeros_blk, nchunks):
    feat = 128
    acc_rows = (NR + 8) * nchunks
    drain_rows = (NR // 16) * nchunks

    mesh = plsc.VectorSubcoreMesh(core_axis_name="c", subcore_axis_name="s")
    cp = pltpu.CompilerParams()
    if "needs_layout_passes" in pltpu.CompilerParams.__dataclass_fields__:
        cp = dataclasses.replace(cp, needs_layout_passes=False)

    @functools.partial(
        pl.kernel,
        out_type=jax.ShapeDtypeStruct((N_PAD * nchunks, feat), jnp.float32),
        mesh=mesh,
        compiler_params=cp,
        scratch_types=[
            pltpu.VMEM_SHARED((acc_rows, feat), jnp.float32),
            pltpu.VMEM((G, feat), jnp.float32),
            pltpu.VMEM((G,), jnp.int32),
            pltpu.VMEM((G,), jnp.int32),
            pltpu.VMEM((16,), jnp.int32),
        ],
    )
    def k(vals_hbm, src_hbm, scat_hbm, broff_hbm, zero_hbm, out_hbm,
          acc, gbuf, sidx, didx, broffs):
        core = jax.lax.axis_index("c")
        sid = jax.lax.axis_index("s")
        pltpu.sync_copy(broff_hbm, broffs)
        lanes = jax.lax.iota(jnp.int32, 16)
        bvec = broffs[...]

        @pl.loop(0, R // 2)
        def _range(i):
            r = 2 * i + core
            nbase = r * NR * nchunks
            b0all = jnp.sum(jnp.where(lanes == r, bvec, 0))
            b1all = jnp.sum(jnp.where(lanes == r + 1, bvec, 0))
            nb = b1all - b0all
            bt0 = b0all + (nb * sid) // 16
            bt1 = b0all + (nb * (sid + 1)) // 16

            pltpu.sync_copy(zero_hbm, acc.at[pl.ds(sid * drain_rows, drain_rows)])
            plsc.subcore_barrier()

            @pl.loop(0, bt1 - bt0)
            def _blk(j):
                e = (bt0 + j) * G
                pltpu.sync_copy(src_hbm.at[pl.ds(e, G)], sidx)
                pltpu.sync_copy(scat_hbm.at[pl.ds(e, G)], didx)
                pltpu.sync_copy(vals_hbm.at[sidx], gbuf)
                pltpu.sync_copy(gbuf, acc.at[didx], add=True)

            plsc.subcore_barrier()
            pltpu.sync_copy(
                acc.at[pl.ds(sid * drain_rows, drain_rows)],
                out_hbm.at[pl.ds(nbase + sid * drain_rows, drain_rows)],
            )
            plsc.subcore_barrier()

    return k(values2d, src_x, scat_x, broff, zeros_blk)


def _layer_tc(aggsum, xin, Wl, bl, Wr, off_lo, off_hi, want_relu, two_out):
    n, cin = xin.shape
    cout = Wl.shape[1]
    grid = n // BR

    def body(a_ref, x_ref, wl_ref, bl_ref, wr_ref, lo_ref, hi_ref, *outs):
        cnt = (hi_ref[...] - lo_ref[...]).astype(jnp.float32)
        inv = 1.0 / jnp.maximum(cnt, 1.0)
        a = a_ref[...] * inv
        out = (jnp.dot(a, wl_ref[...], preferred_element_type=jnp.float32)
               + bl_ref[...]
               + jnp.dot(x_ref[...], wr_ref[...],
                         preferred_element_type=jnp.float32))
        if two_out:
            outs[0][...] = out
            outs[1][...] = jnp.maximum(out, 0.0)
        elif want_relu:
            outs[0][...] = jnp.maximum(out, 0.0)
        else:
            outs[0][...] = out

    out_shape = jax.ShapeDtypeStruct((n, cout), jnp.float32)
    out_shapes = (out_shape, out_shape) if two_out else out_shape
    out_spec = pl.BlockSpec((BR, cout), lambda i: (i, 0))
    out_specs = (out_spec, out_spec) if two_out else out_spec

    return pl.pallas_call(
        body,
        grid=(grid,),
        in_specs=[
            pl.BlockSpec((BR, cin), lambda i: (i, 0)),
            pl.BlockSpec((BR, cin), lambda i: (i, 0)),
            pl.BlockSpec((cin, cout), lambda i: (0, 0)),
            pl.BlockSpec((1, cout), lambda i: (0, 0)),
            pl.BlockSpec((cin, cout), lambda i: (0, 0)),
            pl.BlockSpec((BR, 1), lambda i: (i, 0)),
            pl.BlockSpec((BR, 1), lambda i: (i, 0)),
        ],
        out_specs=out_specs,
        out_shape=out_shapes,
    )(aggsum, xin, Wl, bl, Wr, off_lo, off_hi)


def _mm2_tc(h2, Wl2p, Wr2p):
    n, cin = h2.shape
    cout = Wl2p.shape[1]
    grid = n // BR

    def body(h_ref, wl_ref, wr_ref, y_ref, z_ref):
        h = h_ref[...]
        y_ref[...] = jnp.dot(h, wl_ref[...], preferred_element_type=jnp.float32)
        z_ref[...] = jnp.dot(h, wr_ref[...], preferred_element_type=jnp.float32)

    shp = jax.ShapeDtypeStruct((n, cout), jnp.float32)
    spec = pl.BlockSpec((BR, cout), lambda i: (i, 0))
    return pl.pallas_call(
        body,
        grid=(grid,),
        in_specs=[
            pl.BlockSpec((BR, cin), lambda i: (i, 0)),
            pl.BlockSpec((cin, cout), lambda i: (0, 0)),
            pl.BlockSpec((cin, cout), lambda i: (0, 0)),
        ],
        out_specs=(spec, spec),
        out_shape=(shp, shp),
    )(h2, Wl2p, Wr2p)


def _final_tc(agg2sum, z2, bl2p, off_lo, off_hi, valid):
    n, cout = z2.shape
    grid = n // BR

    def body(a_ref, z_ref, bl_ref, lo_ref, hi_ref, logits_ref, logp_ref):
        cnt = (hi_ref[...] - lo_ref[...]).astype(jnp.float32)
        inv = 1.0 / jnp.maximum(cnt, 1.0)
        logits = a_ref[...] * inv + bl_ref[...] + z_ref[...]
        logits_ref[...] = logits
        col = jax.lax.broadcasted_iota(jnp.int32, (BR, cout), 1)
        mask = col < valid
        neg = jnp.float32(-1e30)
        lm = jnp.where(mask, logits, neg)
        mx = jnp.max(lm, axis=1, keepdims=True)
        ex = jnp.where(mask, jnp.exp(logits - mx), 0.0)
        lse = jnp.log(jnp.sum(ex, axis=1, keepdims=True))
        logp_ref[...] = logits - mx - lse

    shp = jax.ShapeDtypeStruct((n, cout), jnp.float32)
    spec = pl.BlockSpec((BR, cout), lambda i: (i, 0))
    return pl.pallas_call(
        body,
        grid=(grid,),
        in_specs=[
            pl.BlockSpec((BR, cout), lambda i: (i, 0)),
            pl.BlockSpec((BR, cout), lambda i: (i, 0)),
            pl.BlockSpec((1, cout), lambda i: (0, 0)),
            pl.BlockSpec((BR, 1), lambda i: (i, 0)),
            pl.BlockSpec((BR, 1), lambda i: (i, 0)),
        ],
        out_specs=(spec, spec),
        out_shape=(shp, shp),
    )(agg2sum, z2, bl2p, off_lo, off_hi)


def kernel(x, edge_index, Wl0, bl0, Wr0, Wl1, bl1, Wr1, Wl2, bl2, Wr2):
    src = edge_index[0]
    dst = edge_index[1]

    order = jnp.argsort(dst)
    dst_s = dst[order]
    src_s = src[order]
    offsets = jnp.searchsorted(
        dst_s, jnp.arange(N_NODES + 1, dtype=jnp.int32)).astype(jnp.int32)
    roff = jnp.searchsorted(
        dst_s, jnp.arange(R + 1, dtype=jnp.int32) * NR).astype(jnp.int32)
    len_r = roff[1:] - roff[:-1]
    nb_r = (len_r + GE - 1) // GE
    newstart = jnp.concatenate(
        [jnp.zeros((1,), jnp.int32), jnp.cumsum(nb_r).astype(jnp.int32)]) * GE
    rp = dst_s // NR
    newpos = newstart[rp] + (jnp.arange(N_EDGES, dtype=jnp.int32) - roff[rp])
    src_pk = jnp.zeros((E_NEW,), jnp.int32).at[newpos].set(src_s)
    scat_pk = jnp.full((E_NEW,), NR, jnp.int32).at[newpos].set(dst_s % NR)
    off_lo = offsets[:N_NODES].reshape(N_NODES, 1)
    off_hi = offsets[1:].reshape(N_NODES, 1)

    def expand(nchunks):
        cix = jnp.arange(nchunks, dtype=jnp.int32)
        s = (src_pk[:, None] * nchunks + cix).reshape(-1)
        d = (scat_pk[:, None] * nchunks + cix).reshape(-1)
        b = jnp.pad((newstart * nchunks) // G, (0, 16 - (R + 1)))
        z = jnp.zeros(((NR // 16) * nchunks, 128), jnp.float32)
        return s, d, b, z

    src1, scat1, broff1, z1 = expand(1)
    src2, scat2, broff2, z2 = expand(2)
    src4, scat4, broff4, z4 = expand(4)

    bl0r = bl0.reshape(1, -1)
    bl1r = bl1.reshape(1, -1)
    Wl2p = jnp.pad(Wl2, ((0, 0), (0, 128 - 47)))
    Wr2p = jnp.pad(Wr2, ((0, 0), (0, 128 - 47)))
    bl2p = jnp.pad(bl2, (0, 128 - 47)).reshape(1, -1)

    agg0 = _sc_segsum(x.reshape(-1, 128), src2, scat2, broff2, z2, 2)
    agg0 = agg0.reshape(N_PAD, 256)
    h = _layer_tc(agg0[:N_NODES], x, Wl0, bl0r, Wr0, off_lo, off_hi,
                  want_relu=True, two_out=False)

    agg1 = _sc_segsum(h.reshape(-1, 128), src4, scat4, broff4, z4, 4)
    agg1 = agg1.reshape(N_PAD, 512)
    out2, h2 = _layer_tc(agg1[:N_NODES], h, Wl1, bl1r, Wr1, off_lo, off_hi,
                         want_relu=True, two_out=True)

    y2, z2row = _mm2_tc(h2, Wl2p, Wr2p)
    agg2 = _sc_segsum(y2, src1, scat1, broff1, z1, 1)
    logits_p, logp_p = _final_tc(agg2[:N_NODES], z2row, bl2p, off_lo, off_hi,
                                 47)

    logits = logits_p[:, :47]
    logp = logp_p[:, :47]
    return (logp, out2, h2, logits)


# speedup vs baseline: 1.5136x; 1.0968x over previous
import dataclasses
import functools
import jax
import jax.numpy as jnp
from jax.experimental import pallas as pl
from jax.experimental.pallas import tpu as pltpu
from jax.experimental.pallas import tpu_sc as plsc

N_NODES = 10000
N_EDGES = 160000
NR = 1024
R = 10
N_PAD = NR * R
GE = 128
G = 128
E_NEW = N_EDGES + R * GE
BR = 400


def _sc_segsum(values2d, src_x, scat_x, broff, zeros_blk, nchunks):
    feat = 128
    acc_rows = (NR + 8) * nchunks
    drain_rows = (NR // 16) * nchunks

    mesh = plsc.VectorSubcoreMesh(core_axis_name="c", subcore_axis_name="s")
    cp = pltpu.CompilerParams()
    if "needs_layout_passes" in pltpu.CompilerParams.__dataclass_fields__:
        cp = dataclasses.replace(cp, needs_layout_passes=False)

    @functools.partial(
        pl.kernel,
        out_type=jax.ShapeDtypeStruct((N_PAD * nchunks, feat), jnp.float32),
        mesh=mesh,
        compiler_params=cp,
        scratch_types=[
            pltpu.VMEM_SHARED((acc_rows, feat), jnp.float32),
            pltpu.VMEM((2, G, feat), jnp.float32),
            pltpu.VMEM((4, G), jnp.int32),
            pltpu.VMEM((4, G), jnp.int32),
            pltpu.VMEM((16,), jnp.int32),
            pltpu.SemaphoreType.DMA((4,)),
            pltpu.SemaphoreType.DMA((2,)),
            pltpu.SemaphoreType.DMA((2,)),
        ],
    )
    def k(vals_hbm, src_hbm, scat_hbm, broff_hbm, zero_hbm, out_hbm,
          acc, gbuf, sidx, didx, broffs, sem_i, sem_g, sem_s):
        core = jax.lax.axis_index("c")
        sid = jax.lax.axis_index("s")
        pltpu.sync_copy(broff_hbm, broffs)
        lanes = jax.lax.iota(jnp.int32, 16)
        bvec = broffs[...]

        @pl.loop(0, R // 2)
        def _range(i):
            r = 2 * i + core
            nbase = r * NR * nchunks
            b0all = jnp.sum(jnp.where(lanes == r, bvec, 0))
            b1all = jnp.sum(jnp.where(lanes == r + 1, bvec, 0))
            nb = b1all - b0all
            bt0 = b0all + (nb * sid) // 16
            bt1 = b0all + (nb * (sid + 1)) // 16
            nblk = bt1 - bt0

            pltpu.sync_copy(zero_hbm, acc.at[pl.ds(sid * drain_rows, drain_rows)])
            plsc.subcore_barrier()

            def fire_idx(j, slot):
                e = (bt0 + j) * G
                pltpu.async_copy(src_hbm.at[pl.ds(e, G)], sidx.at[slot],
                                 sem_i.at[slot])
                pltpu.async_copy(scat_hbm.at[pl.ds(e, G)], didx.at[slot],
                                 sem_i.at[slot])

            def wait_idx(slot):
                pltpu.make_async_copy(src_hbm.at[pl.ds(0, G)], sidx.at[slot],
                                      sem_i.at[slot]).wait()
                pltpu.make_async_copy(scat_hbm.at[pl.ds(0, G)], didx.at[slot],
                                      sem_i.at[slot]).wait()

            def fire_gather(slot4, b2):
                pltpu.async_copy(vals_hbm.at[sidx.at[slot4]], gbuf.at[b2],
                                 sem_g.at[b2])

            def wait_gather(slot4, b2):
                pltpu.make_async_copy(vals_hbm.at[sidx.at[slot4]],
                                      gbuf.at[b2], sem_g.at[b2]).wait()

            def fire_scat(slot4, b2):
                pltpu.async_copy(gbuf.at[b2], acc.at[didx.at[slot4]],
                                 sem_s.at[b2], add=True)

            def wait_scat(slot4, b2):
                pltpu.make_async_copy(gbuf.at[b2], acc.at[didx.at[slot4]],
                                      sem_s.at[b2]).wait()

            @pl.when(nblk > 0)
            def _():
                fire_idx(0, 0)

            @pl.when(nblk > 1)
            def _():
                fire_idx(1, 1)

            @pl.loop(0, (nblk + 3) // 4)
            def _quad(jo):
                for u in range(4):
                    j = jo * 4 + u
                    b2 = u % 2

                    @pl.when(j < nblk)
                    def _():
                        @pl.when(j >= 2)
                        def _():
                            wait_scat((u + 2) % 4, b2)
                        wait_idx(u)
                        fire_gather(u, b2)

                        @pl.when(j + 2 < nblk)
                        def _():
                            fire_idx(j + 2, (u + 2) % 4)

                        @pl.when(j >= 1)
                        def _():
                            wait_gather((u + 3) % 4, 1 - b2)
                            fire_scat((u + 3) % 4, 1 - b2)

            for m in range(4):
                @pl.when((nblk > 0) & (nblk % 4 == m))
                def _():
                    lu = (m + 3) % 4
                    lb = (nblk - 1) % 2
                    wait_gather(lu, (m + 1) % 2)
                    fire_scat(lu, (m + 1) % 2)

                    @pl.when(nblk >= 2)
                    def _():
                        wait_scat((m + 2) % 4, m % 2)
                    wait_scat(lu, (m + 1) % 2)

            plsc.subcore_barrier()
            pltpu.sync_copy(
                acc.at[pl.ds(sid * drain_rows, drain_rows)],
                out_hbm.at[pl.ds(nbase + sid * drain_rows, drain_rows)],
            )
            plsc.subcore_barrier()

    return k(values2d, src_x, scat_x, broff, zeros_blk)


def _layer_tc(aggsum, xin, Wl, bl, Wr, off_lo, off_hi, want_relu, two_out):
    n, cin = xin.shape
    cout = Wl.shape[1]
    grid = n // BR

    def body(a_ref, x_ref, wl_ref, bl_ref, wr_ref, lo_ref, hi_ref, *outs):
        cnt = (hi_ref[...] - lo_ref[...]).astype(jnp.float32)
        inv = 1.0 / jnp.maximum(cnt, 1.0)
        a = a_ref[...] * inv
        out = (jnp.dot(a, wl_ref[...], preferred_element_type=jnp.float32)
               + bl_ref[...]
               + jnp.dot(x_ref[...], wr_ref[...],
                         preferred_element_type=jnp.float32))
        if two_out:
            outs[0][...] = out
            outs[1][...] = jnp.maximum(out, 0.0)
        elif want_relu:
            outs[0][...] = jnp.maximum(out, 0.0)
        else:
            outs[0][...] = out

    out_shape = jax.ShapeDtypeStruct((n, cout), jnp.float32)
    out_shapes = (out_shape, out_shape) if two_out else out_shape
    out_spec = pl.BlockSpec((BR, cout), lambda i: (i, 0))
    out_specs = (out_spec, out_spec) if two_out else out_spec

    return pl.pallas_call(
        body,
        grid=(grid,),
        in_specs=[
            pl.BlockSpec((BR, cin), lambda i: (i, 0)),
            pl.BlockSpec((BR, cin), lambda i: (i, 0)),
            pl.BlockSpec((cin, cout), lambda i: (0, 0)),
            pl.BlockSpec((1, cout), lambda i: (0, 0)),
            pl.BlockSpec((cin, cout), lambda i: (0, 0)),
            pl.BlockSpec((BR, 1), lambda i: (i, 0)),
            pl.BlockSpec((BR, 1), lambda i: (i, 0)),
        ],
        out_specs=out_specs,
        out_shape=out_shapes,
    )(aggsum, xin, Wl, bl, Wr, off_lo, off_hi)


def _mm2_tc(h2, Wl2p, Wr2p):
    n, cin = h2.shape
    cout = Wl2p.shape[1]
    grid = n // BR

    def body(h_ref, wl_ref, wr_ref, y_ref, z_ref):
        h = h_ref[...]
        y_ref[...] = jnp.dot(h, wl_ref[...], preferred_element_type=jnp.float32)
        z_ref[...] = jnp.dot(h, wr_ref[...], preferred_element_type=jnp.float32)

    shp = jax.ShapeDtypeStruct((n, cout), jnp.float32)
    spec = pl.BlockSpec((BR, cout), lambda i: (i, 0))
    return pl.pallas_call(
        body,
        grid=(grid,),
        in_specs=[
            pl.BlockSpec((BR, cin), lambda i: (i, 0)),
            pl.BlockSpec((cin, cout), lambda i: (0, 0)),
            pl.BlockSpec((cin, cout), lambda i: (0, 0)),
        ],
        out_specs=(spec, spec),
        out_shape=(shp, shp),
    )(h2, Wl2p, Wr2p)


def _final_tc(agg2sum, z2, bl2p, off_lo, off_hi, valid):
    n, cout = z2.shape
    grid = n // BR

    def body(a_ref, z_ref, bl_ref, lo_ref, hi_ref, logits_ref, logp_ref):
        cnt = (hi_ref[...] - lo_ref[...]).astype(jnp.float32)
        inv = 1.0 / jnp.maximum(cnt, 1.0)
        logits = a_ref[...] * inv + bl_ref[...] + z_ref[...]
        logits_ref[...] = logits
        col = jax.lax.broadcasted_iota(jnp.int32, (BR, cout), 1)
        mask = col < valid
        neg = jnp.float32(-1e30)
        lm = jnp.where(mask, logits, neg)
        mx = jnp.max(lm, axis=1, keepdims=True)
        ex = jnp.where(mask, jnp.exp(logits - mx), 0.0)
        lse = jnp.log(jnp.sum(ex, axis=1, keepdims=True))
        logp_ref[...] = logits - mx - lse

    shp = jax.ShapeDtypeStruct((n, cout), jnp.float32)
    spec = pl.BlockSpec((BR, cout), lambda i: (i, 0))
    return pl.pallas_call(
        body,
        grid=(grid,),
        in_specs=[
            pl.BlockSpec((BR, cout), lambda i: (i, 0)),
            pl.BlockSpec((BR, cout), lambda i: (i, 0)),
            pl.BlockSpec((1, cout), lambda i: (0, 0)),
            pl.BlockSpec((BR, 1), lambda i: (i, 0)),
            pl.BlockSpec((BR, 1), lambda i: (i, 0)),
        ],
        out_specs=(spec, spec),
        out_shape=(shp, shp),
    )(agg2sum, z2, bl2p, off_lo, off_hi)


def kernel(x, edge_index, Wl0, bl0, Wr0, Wl1, bl1, Wr1, Wl2, bl2, Wr2):
    src = edge_index[0]
    dst = edge_index[1]

    order = jnp.argsort(dst)
    dst_s = dst[order]
    src_s = src[order]
    offsets = jnp.searchsorted(
        dst_s, jnp.arange(N_NODES + 1, dtype=jnp.int32)).astype(jnp.int32)
    roff = jnp.searchsorted(
        dst_s, jnp.arange(R + 1, dtype=jnp.int32) * NR).astype(jnp.int32)
    len_r = roff[1:] - roff[:-1]
    nb_r = (len_r + GE - 1) // GE
    newstart = jnp.concatenate(
        [jnp.zeros((1,), jnp.int32), jnp.cumsum(nb_r).astype(jnp.int32)]) * GE
    rp = dst_s // NR
    newpos = newstart[rp] + (jnp.arange(N_EDGES, dtype=jnp.int32) - roff[rp])
    src_pk = jnp.zeros((E_NEW,), jnp.int32).at[newpos].set(src_s)
    scat_pk = jnp.full((E_NEW,), NR, jnp.int32).at[newpos].set(dst_s % NR)
    off_lo = offsets[:N_NODES].reshape(N_NODES, 1)
    off_hi = offsets[1:].reshape(N_NODES, 1)

    def expand(nchunks):
        cix = jnp.arange(nchunks, dtype=jnp.int32)
        s = (src_pk[:, None] * nchunks + cix).reshape(-1)
        d = (scat_pk[:, None] * nchunks + cix).reshape(-1)
        b = jnp.pad((newstart * nchunks) // G, (0, 16 - (R + 1)))
        z = jnp.zeros(((NR // 16) * nchunks, 128), jnp.float32)
        return s, d, b, z

    src1, scat1, broff1, z1 = expand(1)
    src2, scat2, broff2, z2 = expand(2)
    src4, scat4, broff4, z4 = expand(4)

    bl0r = bl0.reshape(1, -1)
    bl1r = bl1.reshape(1, -1)
    Wl2p = jnp.pad(Wl2, ((0, 0), (0, 128 - 47)))
    Wr2p = jnp.pad(Wr2, ((0, 0), (0, 128 - 47)))
    bl2p = jnp.pad(bl2, (0, 128 - 47)).reshape(1, -1)

    agg0 = _sc_segsum(x.reshape(-1, 128), src2, scat2, broff2, z2, 2)
    agg0 = agg0.reshape(N_PAD, 256)
    h = _layer_tc(agg0[:N_NODES], x, Wl0, bl0r, Wr0, off_lo, off_hi,
                  want_relu=True, two_out=False)

    agg1 = _sc_segsum(h.reshape(-1, 128), src4, scat4, broff4, z4, 4)
    agg1 = agg1.reshape(N_PAD, 512)
    out2, h2 = _layer_tc(agg1[:N_NODES], h, Wl1, bl1r, Wr1, off_lo, off_hi,
                         want_relu=True, two_out=True)

    y2, z2row = _mm2_tc(h2, Wl2p, Wr2p)
    agg2 = _sc_segsum(y2, src1, scat1, broff1, z1, 1)
    logits_p, logp_p = _final_tc(agg2[:N_NODES], z2row, bl2p, off_lo, off_hi,
                                 47)

    logits = logits_p[:, :47]
    logp = logp_p[:, :47]
    return (logp, out2, h2, logits)


# speedup vs baseline: 2.2128x; 1.4619x over previous
import dataclasses
import functools
import jax
import jax.numpy as jnp
from jax.experimental import pallas as pl
from jax.experimental.pallas import tpu as pltpu
from jax.experimental.pallas import tpu_sc as plsc

N_NODES = 10000
N_EDGES = 160000
NR = 1024
R = 10
N_PAD = NR * R
GE = 128
G = 128
E_NEW = N_EDGES + R * GE
BR = 400


def _sc_segsum(values2d, src_x, scat_x, broff, zeros_blk, nchunks):
    feat = 128
    acc_rows = (NR + 8) * nchunks
    drain_rows = (NR // 16) * nchunks

    mesh = plsc.VectorSubcoreMesh(core_axis_name="c", subcore_axis_name="s")
    cp = pltpu.CompilerParams()
    if "needs_layout_passes" in pltpu.CompilerParams.__dataclass_fields__:
        cp = dataclasses.replace(cp, needs_layout_passes=False)

    @functools.partial(
        pl.kernel,
        out_type=jax.ShapeDtypeStruct((N_PAD * nchunks, feat), jnp.float32),
        mesh=mesh,
        compiler_params=cp,
        scratch_types=[
            pltpu.VMEM_SHARED((acc_rows, feat), jnp.float32),
            pltpu.VMEM((2, G, feat), jnp.float32),
            pltpu.VMEM((4, G), jnp.int32),
            pltpu.VMEM((4, G), jnp.int32),
            pltpu.VMEM((16,), jnp.int32),
            pltpu.SemaphoreType.DMA((4,)),
            pltpu.SemaphoreType.DMA((2,)),
            pltpu.SemaphoreType.DMA((2,)),
        ],
    )
    def k(vals_hbm, src_hbm, scat_hbm, broff_hbm, zero_hbm, out_hbm,
          acc, gbuf, sidx, didx, broffs, sem_i, sem_g, sem_s):
        core = jax.lax.axis_index("c")
        sid = jax.lax.axis_index("s")
        pltpu.sync_copy(broff_hbm, broffs)
        lanes = jax.lax.iota(jnp.int32, 16)
        bvec = broffs[...]

        @pl.loop(0, R // 2)
        def _range(i):
            r = 2 * i + core
            nbase = r * NR * nchunks
            b0all = jnp.sum(jnp.where(lanes == r, bvec, 0))
            b1all = jnp.sum(jnp.where(lanes == r + 1, bvec, 0))
            nb = b1all - b0all
            bt0 = b0all + (nb * sid) // 16
            bt1 = b0all + (nb * (sid + 1)) // 16
            nblk = bt1 - bt0

            pltpu.sync_copy(zero_hbm, acc.at[pl.ds(sid * drain_rows, drain_rows)])
            plsc.subcore_barrier()

            def fire_idx(j, slot):
                e = (bt0 + j) * G
                pltpu.async_copy(src_hbm.at[pl.ds(e, G)], sidx.at[slot],
                                 sem_i.at[slot])
                pltpu.async_copy(scat_hbm.at[pl.ds(e, G)], didx.at[slot],
                                 sem_i.at[slot])

            def wait_idx(slot):
                pltpu.make_async_copy(src_hbm.at[pl.ds(0, G)], sidx.at[slot],
                                      sem_i.at[slot]).wait()
                pltpu.make_async_copy(scat_hbm.at[pl.ds(0, G)], didx.at[slot],
                                      sem_i.at[slot]).wait()

            def fire_gather(slot4, b2):
                pltpu.async_copy(vals_hbm.at[sidx.at[slot4]], gbuf.at[b2],
                                 sem_g.at[b2])

            def wait_gather(slot4, b2):
                pltpu.make_async_copy(vals_hbm.at[sidx.at[slot4]],
                                      gbuf.at[b2], sem_g.at[b2]).wait()

            def fire_scat(slot4, b2):
                pltpu.async_copy(gbuf.at[b2], acc.at[didx.at[slot4]],
                                 sem_s.at[b2], add=True)

            def wait_scat(slot4, b2):
                pltpu.make_async_copy(gbuf.at[b2], acc.at[didx.at[slot4]],
                                      sem_s.at[b2]).wait()

            @pl.when(nblk > 0)
            def _():
                fire_idx(0, 0)

            @pl.when(nblk > 1)
            def _():
                fire_idx(1, 1)

            @pl.loop(0, (nblk + 3) // 4)
            def _quad(jo):
                for u in range(4):
                    j = jo * 4 + u
                    b2 = u % 2

                    @pl.when(j < nblk)
                    def _():
                        @pl.when(j >= 2)
                        def _():
                            wait_scat((u + 2) % 4, b2)
                        wait_idx(u)
                        fire_gather(u, b2)

                        @pl.when(j + 2 < nblk)
                        def _():
                            fire_idx(j + 2, (u + 2) % 4)

                        @pl.when(j >= 1)
                        def _():
                            wait_gather((u + 3) % 4, 1 - b2)
                            fire_scat((u + 3) % 4, 1 - b2)

            for m in range(4):
                @pl.when((nblk > 0) & (nblk % 4 == m))
                def _():
                    lu = (m + 3) % 4
                    lb = (nblk - 1) % 2
                    wait_gather(lu, (m + 1) % 2)
                    fire_scat(lu, (m + 1) % 2)

                    @pl.when(nblk >= 2)
                    def _():
                        wait_scat((m + 2) % 4, m % 2)
                    wait_scat(lu, (m + 1) % 2)

            plsc.subcore_barrier()
            pltpu.sync_copy(
                acc.at[pl.ds(sid * drain_rows, drain_rows)],
                out_hbm.at[pl.ds(nbase + sid * drain_rows, drain_rows)],
            )
            plsc.subcore_barrier()

    return k(values2d, src_x, scat_x, broff, zeros_blk)


def _layer_tc(aggsum, xin, Wl, bl, Wr, cnt, want_relu, two_out):
    n, cin = xin.shape
    cout = Wl.shape[1]
    grid = n // BR

    def body(a_ref, x_ref, wl_ref, bl_ref, wr_ref, cnt_ref, *outs):
        cnt = cnt_ref[...].astype(jnp.float32)
        inv = 1.0 / jnp.maximum(cnt, 1.0)
        a = a_ref[...] * inv
        out = (jnp.dot(a, wl_ref[...], preferred_element_type=jnp.float32)
               + bl_ref[...]
               + jnp.dot(x_ref[...], wr_ref[...],
                         preferred_element_type=jnp.float32))
        if two_out:
            outs[0][...] = out
            outs[1][...] = jnp.maximum(out, 0.0)
        elif want_relu:
            outs[0][...] = jnp.maximum(out, 0.0)
        else:
            outs[0][...] = out

    out_shape = jax.ShapeDtypeStruct((n, cout), jnp.float32)
    out_shapes = (out_shape, out_shape) if two_out else out_shape
    out_spec = pl.BlockSpec((BR, cout), lambda i: (i, 0))
    out_specs = (out_spec, out_spec) if two_out else out_spec

    return pl.pallas_call(
        body,
        grid=(grid,),
        in_specs=[
            pl.BlockSpec((BR, cin), lambda i: (i, 0)),
            pl.BlockSpec((BR, cin), lambda i: (i, 0)),
            pl.BlockSpec((cin, cout), lambda i: (0, 0)),
            pl.BlockSpec((1, cout), lambda i: (0, 0)),
            pl.BlockSpec((cin, cout), lambda i: (0, 0)),
            pl.BlockSpec((BR, 1), lambda i: (i, 0)),
        ],
        out_specs=out_specs,
        out_shape=out_shapes,
    )(aggsum, xin, Wl, bl, Wr, cnt)


def _mm2_tc(h2, Wl2p, Wr2p):
    n, cin = h2.shape
    cout = Wl2p.shape[1]
    grid = n // BR

    def body(h_ref, wl_ref, wr_ref, y_ref, z_ref):
        h = h_ref[...]
        y_ref[...] = jnp.dot(h, wl_ref[...], preferred_element_type=jnp.float32)
        z_ref[...] = jnp.dot(h, wr_ref[...], preferred_element_type=jnp.float32)

    shp = jax.ShapeDtypeStruct((n, cout), jnp.float32)
    spec = pl.BlockSpec((BR, cout), lambda i: (i, 0))
    return pl.pallas_call(
        body,
        grid=(grid,),
        in_specs=[
            pl.BlockSpec((BR, cin), lambda i: (i, 0)),
            pl.BlockSpec((cin, cout), lambda i: (0, 0)),
            pl.BlockSpec((cin, cout), lambda i: (0, 0)),
        ],
        out_specs=(spec, spec),
        out_shape=(shp, shp),
    )(h2, Wl2p, Wr2p)


def _final_tc(agg2sum, z2, bl2p, cnt_arr, valid):
    n, cout = z2.shape
    grid = n // BR

    def body(a_ref, z_ref, bl_ref, cnt_ref, logits_ref, logp_ref):
        cnt = cnt_ref[...].astype(jnp.float32)
        inv = 1.0 / jnp.maximum(cnt, 1.0)
        logits = a_ref[...] * inv + bl_ref[...] + z_ref[...]
        logits_ref[...] = logits
        col = jax.lax.broadcasted_iota(jnp.int32, (BR, cout), 1)
        mask = col < valid
        neg = jnp.float32(-1e30)
        lm = jnp.where(mask, logits, neg)
        mx = jnp.max(lm, axis=1, keepdims=True)
        ex = jnp.where(mask, jnp.exp(logits - mx), 0.0)
        lse = jnp.log(jnp.sum(ex, axis=1, keepdims=True))
        logp_ref[...] = logits - mx - lse

    shp = jax.ShapeDtypeStruct((n, cout), jnp.float32)
    spec = pl.BlockSpec((BR, cout), lambda i: (i, 0))
    return pl.pallas_call(
        body,
        grid=(grid,),
        in_specs=[
            pl.BlockSpec((BR, cout), lambda i: (i, 0)),
            pl.BlockSpec((BR, cout), lambda i: (i, 0)),
            pl.BlockSpec((1, cout), lambda i: (0, 0)),
            pl.BlockSpec((BR, 1), lambda i: (i, 0)),
        ],
        out_specs=(spec, spec),
        out_shape=(shp, shp),
    )(agg2sum, z2, bl2p, cnt_arr)


def kernel(x, edge_index, Wl0, bl0, Wr0, Wl1, bl1, Wr1, Wl2, bl2, Wr2):
    src = edge_index[0]
    dst = edge_index[1]

    cnt = jnp.zeros((N_NODES,), jnp.int32).at[dst].add(1).reshape(N_NODES, 1)
    rp = dst // NR
    oh = (rp[:, None] == jnp.arange(R, dtype=jnp.int32)[None, :])
    pib = jnp.cumsum(oh.astype(jnp.int32), axis=0)
    rank = jnp.take_along_axis(pib, rp[:, None], axis=1)[:, 0] - 1
    len_r = pib[-1]
    nb_r = (len_r + GE - 1) // GE
    newstart = jnp.concatenate(
        [jnp.zeros((1,), jnp.int32), jnp.cumsum(nb_r).astype(jnp.int32)]) * GE
    newpos = newstart[rp] + rank
    src_pk = jnp.zeros((E_NEW,), jnp.int32).at[newpos].set(src)
    scat_pk = jnp.full((E_NEW,), NR, jnp.int32).at[newpos].set(dst % NR)

    def expand(nchunks):
        cix = jnp.arange(nchunks, dtype=jnp.int32)
        s = (src_pk[:, None] * nchunks + cix).reshape(-1)
        d = (scat_pk[:, None] * nchunks + cix).reshape(-1)
        b = jnp.pad((newstart * nchunks) // G, (0, 16 - (R + 1)))
        z = jnp.zeros(((NR // 16) * nchunks, 128), jnp.float32)
        return s, d, b, z

    src1, scat1, broff1, z1 = expand(1)
    src2, scat2, broff2, z2 = expand(2)
    src4, scat4, broff4, z4 = expand(4)

    bl0r = bl0.reshape(1, -1)
    bl1r = bl1.reshape(1, -1)
    Wl2p = jnp.pad(Wl2, ((0, 0), (0, 128 - 47)))
    Wr2p = jnp.pad(Wr2, ((0, 0), (0, 128 - 47)))
    bl2p = jnp.pad(bl2, (0, 128 - 47)).reshape(1, -1)

    agg0 = _sc_segsum(x.reshape(-1, 128), src2, scat2, broff2, z2, 2)
    agg0 = agg0.reshape(N_PAD, 256)
    h = _layer_tc(agg0[:N_NODES], x, Wl0, bl0r, Wr0, cnt,
                  want_relu=True, two_out=False)

    agg1 = _sc_segsum(h.reshape(-1, 128), src4, scat4, broff4, z4, 4)
    agg1 = agg1.reshape(N_PAD, 512)
    out2, h2 = _layer_tc(agg1[:N_NODES], h, Wl1, bl1r, Wr1, cnt,
                         want_relu=True, two_out=True)

    y2, z2row = _mm2_tc(h2, Wl2p, Wr2p)
    agg2 = _sc_segsum(y2, src1, scat1, broff1, z1, 1)
    logits_p, logp_p = _final_tc(agg2[:N_NODES], z2row, bl2p, cnt, 47)

    logits = logits_p[:, :47]
    logp = logp_p[:, :47]
    return (logp, out2, h2, logits)
